# Initial kernel scaffold; baseline (speedup 1.0000x reference)
import jax
import jax.numpy as jnp
from jax.experimental import pallas as pl


def kernel(tensor):
    raise NotImplementedError("write your pallas kernel here")


# speedup vs baseline: 49.6395x; 49.6395x over previous
import functools

import jax
import jax.numpy as jnp
import numpy as np
from jax import lax
from jax.experimental import pallas as pl
from jax.experimental.pallas import tpu as pltpu
from jax.experimental.pallas import tpu_sc as plsc

_H = 224
_NH = 28
_NGROUPS = 16
_GSIZE = 3 * _NH * _NH
_NCHUNK = _GSIZE // 16
_LOG2E = 1.4426950408889634

def _resize_weights(in_size, out_size):
    inv_scale = np.float32(in_size / out_size)
    sample_f = (np.arange(out_size, dtype=np.float32) + np.float32(0.5)) * inv_scale
    sample_f = sample_f - np.float32(0.5)
    x = np.abs(sample_f[None, :] - np.arange(in_size, dtype=np.float32)[:, None])
    w = np.maximum(np.float32(0), np.float32(1) - x / inv_scale).astype(np.float32)
    total = w.sum(axis=0, keepdims=True, dtype=np.float32)
    return (w / total).astype(np.float32)


_W_RESIZE = _resize_weights(_H, _NH)


def _resize_body(x_ref, w_ref, o_ref):
    x = x_ref[...]
    w = w_ref[...]
    hi = jax.lax.Precision.HIGHEST
    y = lax.dot_general(x, w, (((1,), (0,)), ((), ())), precision=hi)
    r = lax.dot_general(y, w, (((1,), (0,)), ((), ())), precision=hi)
    o_ref[0] = r


def _resize_tc(x_flat):
    return pl.pallas_call(
        _resize_body,
        grid=(_NGROUPS,),
        in_specs=[
            pl.BlockSpec((3, _H, _H), lambda p: (4 * p, 0, 0)),
            pl.BlockSpec((_H, _NH), lambda p: (0, 0)),
        ],
        out_specs=pl.BlockSpec((1, 3, _NH, _NH), lambda p: (p, 0, 0, 0)),
        out_shape=jax.ShapeDtypeStruct((_NGROUPS, 3, _NH, _NH), jnp.float32),
    )(x_flat, jnp.asarray(_W_RESIZE))


def _plogp(p):
    bits = plsc.bitcast(p, jnp.int32)
    e_exp = ((bits >> 23) & 0xFF).astype(jnp.float32) - 127.0
    mant = plsc.bitcast((bits & 0x007FFFFF) | 0x3F800000, jnp.float32)
    z = (mant - 1.0) / (mant + 1.0)
    z2 = z * z
    lnm = 2.0 * z * (1.0 + z2 * (1.0 / 3 + z2 * (1.0 / 5 + z2 * (1.0 / 7 + z2 / 9))))
    return p * (e_exp + lnm * _LOG2E)


@functools.cache
def _make_sc_hist_entropy():
    mesh = plsc.VectorSubcoreMesh(core_axis_name="c", subcore_axis_name="s")
    return pl.kernel(
        _sc_body,
        out_type=jax.ShapeDtypeStruct((16 * 16,), jnp.float32),
        mesh=mesh,
        compiler_params=pltpu.CompilerParams(needs_layout_passes=False),
        scratch_types=[
            pltpu.VMEM((_GSIZE,), jnp.float32),
            pltpu.VMEM((512,), jnp.float32),
            pltpu.VMEM((16,), jnp.float32),
        ],
    )


def _sc_body(vals_hbm, out_hbm, vals_v, bins_v, evec_v):
    cid = lax.axis_index("c")
    sid = lax.axis_index("s")

    @pl.when(cid == 0)
    def _work():
        pltpu.sync_copy(vals_hbm.at[pl.ds(sid * _GSIZE, _GSIZE)], vals_v)

        v0 = vals_v[pl.ds(0, 16)]

        def _mm(i, carry):
            vmn, vmx = carry
            v = vals_v[pl.ds(i * 16, 16)]
            return jnp.minimum(vmn, v), jnp.maximum(vmx, v)

        vmn, vmx = lax.fori_loop(1, _NCHUNK, _mm, (v0, v0))
        lane = lax.iota(jnp.int32, 16)

        def _allreduce(v, op):
            for s in (1, 2, 4, 8):
                v = op(v, v.at[lane ^ s].get(mode="promise_in_bounds"))
            return v

        mn = _allreduce(vmn, jnp.minimum)
        mx = _allreduce(vmx, jnp.maximum)
        scale = 30.0 / (mx - mn)

        zero16 = jnp.zeros((16,), jnp.float32)
        for b in range(32):
            bins_v[pl.ds(b * 16, 16)] = zero16
        lane32 = lax.iota(jnp.int32, 16) * 32
        ones16 = jnp.ones((16,), jnp.float32)

        def _hist(i, carry):
            v = vals_v[pl.ds(i * 16, 16)]
            idx = jnp.clip(((v - mn) * scale).astype(jnp.int32), 0, 29)
            plsc.addupdate_scatter(bins_v, [lane32 + idx], ones16)
            return carry

        lax.fori_loop(0, _NCHUNK, _hist, 0)

        h0 = bins_v[pl.ds(0, 16)]
        h1 = bins_v[pl.ds(16, 16)]
        for j in range(1, 16):
            h0 = h0 + bins_v[pl.ds(j * 32, 16)]
            h1 = h1 + bins_v[pl.ds(j * 32 + 16, 16)]

        m = _allreduce(jnp.maximum(h0, h1), jnp.maximum)
        nzero0 = plsc.all_reduce_population_count(h0 == 0.0)
        nzero1 = plsc.all_reduce_population_count(jnp.logical_and(h1 == 0.0, lane < 14))
        anyzero = (nzero0 + nzero1) > 0
        inv = 1.0 / (m + 0.0001)
        ent = -_allreduce(_plogp(h0 * inv) + _plogp(h1 * inv), jnp.add)
        e = jnp.where(anyzero, jnp.float32(1.0), ent)

        evec_v[...] = e
        pltpu.sync_copy(evec_v, out_hbm.at[pl.ds(sid * 16, 16)])


def kernel(tensor):
    x = tensor.reshape(192, _H, _H)
    resized = _resize_tc(x)
    vals = resized.reshape(_NGROUPS * _GSIZE)
    ents = _make_sc_hist_entropy()(vals)
    return jnp.sum(ents.reshape(_NGROUPS, 16)[:, 0]) * jnp.float32(1.0 / 160.0)


# speedup vs baseline: 50.8591x; 1.0246x over previous
import functools

import jax
import jax.numpy as jnp
import numpy as np
from jax import lax
from jax.experimental import pallas as pl
from jax.experimental.pallas import tpu as pltpu
from jax.experimental.pallas import tpu_sc as plsc

_H = 224
_NH = 28
_NGROUPS = 16
_GSIZE = 3 * _NH * _NH
_NCHUNK = _GSIZE // 16
_GSTRIDE = 3072
_LOG2E = 1.4426950408889634

def _resize_weights(in_size, out_size):
    inv_scale = np.float32(in_size / out_size)
    sample_f = (np.arange(out_size, dtype=np.float32) + np.float32(0.5)) * inv_scale
    sample_f = sample_f - np.float32(0.5)
    x = np.abs(sample_f[None, :] - np.arange(in_size, dtype=np.float32)[:, None])
    w = np.maximum(np.float32(0), np.float32(1) - x / inv_scale).astype(np.float32)
    total = w.sum(axis=0, keepdims=True, dtype=np.float32)
    return (w / total).astype(np.float32)


_W_RESIZE = _resize_weights(_H, _NH)


def _resize_body(x_ref, w_ref, o_ref):
    x = x_ref[...]
    w = w_ref[...]
    hi = jax.lax.Precision.HIGHEST
    y = lax.dot_general(x, w, (((2,), (0,)), ((), ())), precision=hi)
    r = lax.dot_general(y, w, (((1,), (0,)), ((), ())), precision=hi)
    r2 = r.reshape(3 * _NH, _NH)
    for i in range(3 * _NH):
        o_ref[pl.ds(i * _NH, _NH)] = r2[i]


def _resize_tc(x_flat):
    return pl.pallas_call(
        _resize_body,
        grid=(_NGROUPS,),
        in_specs=[
            pl.BlockSpec((3, _H, _H), lambda p: (4 * p, 0, 0)),
            pl.BlockSpec((_H, _NH), lambda p: (0, 0)),
        ],
        out_specs=pl.BlockSpec((_GSTRIDE,), lambda p: (p,)),
        out_shape=jax.ShapeDtypeStruct((_NGROUPS * _GSTRIDE,), jnp.float32),
    )(x_flat, jnp.asarray(_W_RESIZE))


def _plogp(p):
    bits = plsc.bitcast(p, jnp.int32)
    e_exp = ((bits >> 23) & 0xFF).astype(jnp.float32) - 127.0
    mant = plsc.bitcast((bits & 0x007FFFFF) | 0x3F800000, jnp.float32)
    z = (mant - 1.0) / (mant + 1.0)
    z2 = z * z
    lnm = 2.0 * z * (1.0 + z2 * (1.0 / 3 + z2 * (1.0 / 5 + z2 * (1.0 / 7 + z2 / 9))))
    return p * (e_exp + lnm * _LOG2E)


@functools.cache
def _make_sc_hist_entropy():
    mesh = plsc.VectorSubcoreMesh(core_axis_name="c", subcore_axis_name="s")
    return pl.kernel(
        _sc_body,
        out_type=jax.ShapeDtypeStruct((_NGROUPS * 16,), jnp.float32),
        mesh=mesh,
        compiler_params=pltpu.CompilerParams(needs_layout_passes=False),
        scratch_types=[
            pltpu.VMEM((_GSIZE,), jnp.float32),
            pltpu.VMEM((512,), jnp.float32),
            pltpu.VMEM((16,), jnp.float32),
        ],
    )


def _sc_body(vals_hbm, out_hbm, vals_v, bins_v, evec_v):
    cid = lax.axis_index("c")
    sid = lax.axis_index("s")

    @pl.when(cid == 0)
    def _work():
        pltpu.sync_copy(vals_hbm.at[pl.ds(sid * _GSTRIDE, _GSIZE)], vals_v)

        v0 = vals_v[pl.ds(0, 16)]

        def _mm(i, carry):
            vmn, vmx = carry
            v = vals_v[pl.ds(i * 16, 16)]
            return jnp.minimum(vmn, v), jnp.maximum(vmx, v)

        vmn, vmx = lax.fori_loop(1, _NCHUNK, _mm, (v0, v0))
        lane = lax.iota(jnp.int32, 16)

        def _allreduce(v, op):
            for s in (1, 2, 4, 8):
                v = op(v, v.at[lane ^ s].get(mode="promise_in_bounds"))
            return v

        mn = _allreduce(vmn, jnp.minimum)
        mx = _allreduce(vmx, jnp.maximum)
        scale = 30.0 / (mx - mn)

        zero16 = jnp.zeros((16,), jnp.float32)
        for b in range(32):
            bins_v[pl.ds(b * 16, 16)] = zero16
        lane32 = lax.iota(jnp.int32, 16) * 32
        ones16 = jnp.ones((16,), jnp.float32)

        def _hist(i, carry):
            v = vals_v[pl.ds(i * 16, 16)]
            idx = jnp.clip(((v - mn) * scale).astype(jnp.int32), 0, 29)
            plsc.addupdate_scatter(bins_v, [lane32 + idx], ones16)
            return carry

        lax.fori_loop(0, _NCHUNK, _hist, 0)

        h0 = bins_v[pl.ds(0, 16)]
        h1 = bins_v[pl.ds(16, 16)]
        for j in range(1, 16):
            h0 = h0 + bins_v[pl.ds(j * 32, 16)]
            h1 = h1 + bins_v[pl.ds(j * 32 + 16, 16)]

        m = _allreduce(jnp.maximum(h0, h1), jnp.maximum)
        nzero0 = plsc.all_reduce_population_count(h0 == 0.0)
        nzero1 = plsc.all_reduce_population_count(jnp.logical_and(h1 == 0.0, lane < 14))
        anyzero = (nzero0 + nzero1) > 0
        inv = 1.0 / (m + 0.0001)
        ent = -_allreduce(_plogp(h0 * inv) + _plogp(h1 * inv), jnp.add)
        e = jnp.where(anyzero, jnp.float32(1.0), ent)

        evec_v[...] = e
        pltpu.sync_copy(evec_v, out_hbm.at[pl.ds(sid * 16, 16)])


def kernel(tensor):
    x = tensor.reshape(192, _H, _H)
    vals = _resize_tc(x)
    ents = _make_sc_hist_entropy()(vals)
    return jnp.sum(ents.reshape(_NGROUPS, 16)[:, 0]) * jnp.float32(1.0 / 160.0)
